# Initial kernel scaffold; baseline (speedup 1.0000x reference)
import jax
import jax.numpy as jnp
from jax.experimental import pallas as pl


def kernel(params, sequence, edge_index):
    raise NotImplementedError("write your pallas kernel here")


# speedup vs baseline: 3.1795x; 3.1795x over previous
import functools

import jax
import jax.numpy as jnp
import numpy as np
from jax import lax
from jax.experimental import pallas as pl
from jax.experimental.pallas import tpu as pltpu
from jax.experimental.pallas import tpu_sc as plsc

N = 10000
NPAD = 10240
EMB = 256
HID = 512
NCHUNK = 4
CW = 128
GCOLS = 3 * HID
GROWS = GCOLS // CW
NTILES = 16
RPT = NPAD // NTILES
BM = 512


def _k0_body(seq_ref, pos_ref, aa_ref, wt_ref, wb_ref, b_ref, o_ref):
    s = seq_ref[...]
    ids = jax.lax.broadcasted_iota(jnp.int32, (BM, 32), 1)
    onehot = (ids == s).astype(jnp.float32)
    t = jnp.dot(aa_ref[...], wt_ref[...], preferred_element_type=jnp.float32)
    aa_part = jnp.dot(onehot, t, preferred_element_type=jnp.float32)
    pos_part = jnp.dot(pos_ref[...], wb_ref[...],
                       preferred_element_type=jnp.float32)
    o_ref[...] = aa_part + pos_part + b_ref[...]


def _input_proj(seq2, pos, aa32, w_top, w_bot, b):
    grid = (NPAD // BM,)
    return pl.pallas_call(
        _k0_body,
        grid=grid,
        in_specs=[
            pl.BlockSpec((BM, 1), lambda r: (r, 0)),
            pl.BlockSpec((BM, EMB), lambda r: (r, 0)),
            pl.BlockSpec((32, EMB), lambda r: (0, 0)),
            pl.BlockSpec((EMB, HID), lambda r: (0, 0)),
            pl.BlockSpec((EMB, HID), lambda r: (0, 0)),
            pl.BlockSpec((1, HID), lambda r: (0, 0)),
        ],
        out_specs=pl.BlockSpec((BM, HID), lambda r: (r, 0)),
        out_shape=jax.ShapeDtypeStruct((NPAD, HID), jnp.float32),
    )(seq2, pos, aa32, w_top, w_bot, b)


def _k1_body(h_ref, w_ref, o_ref):
    o_ref[...] = jnp.dot(h_ref[...], w_ref[...],
                         preferred_element_type=jnp.float32)


def _fused_matmul(h, w3):
    grid = (NPAD // BM,)
    return pl.pallas_call(
        _k1_body,
        grid=grid,
        in_specs=[
            pl.BlockSpec((BM, HID), lambda r: (r, 0)),
            pl.BlockSpec((HID, GCOLS), lambda r: (0, 0)),
        ],
        out_specs=pl.BlockSpec((BM, GCOLS), lambda r: (r, 0)),
        out_shape=jax.ShapeDtypeStruct((NPAD, GCOLS), jnp.float32),
    )(h, w3)


def _k2_body(residual, g_ref, s_ref, deg_ref, h_ref, mb_ref, lb_ref,
             lg_ref, lbn_ref, o_ref):
    g = g_ref[...]
    lin = g[:, :HID]
    a = g[:, HID:2 * HID]
    s = jnp.concatenate([s_ref[c] for c in range(NCHUNK)], axis=-1)
    deg = deg_ref[...][:, 0:1]
    x = lin + lb_ref[...] + deg * (a + mb_ref[...]) + s
    y = jnp.maximum(x, 0.0)
    mu = jnp.mean(y, axis=-1, keepdims=True)
    var = jnp.mean((y - mu) ** 2, axis=-1, keepdims=True)
    ln = (y - mu) / jnp.sqrt(var + 1e-5) * lg_ref[...] + lbn_ref[...]
    o_ref[...] = h_ref[...] + ln if residual else ln


def _combine(residual, g, s, deg, h, msg_b, lin_b, ln_g, ln_b):
    grid = (NPAD // BM,)
    return pl.pallas_call(
        functools.partial(_k2_body, residual),
        grid=grid,
        in_specs=[
            pl.BlockSpec((BM, GCOLS), lambda r: (r, 0)),
            pl.BlockSpec((NCHUNK, BM, CW), lambda r: (0, r, 0)),
            pl.BlockSpec((BM, CW), lambda r: (r, 0)),
            pl.BlockSpec((BM, HID), lambda r: (r, 0)),
            pl.BlockSpec((1, HID), lambda r: (0, 0)),
            pl.BlockSpec((1, HID), lambda r: (0, 0)),
            pl.BlockSpec((1, HID), lambda r: (0, 0)),
            pl.BlockSpec((1, HID), lambda r: (0, 0)),
        ],
        out_specs=pl.BlockSpec((BM, HID), lambda r: (r, 0)),
        out_shape=jax.ShapeDtypeStruct((NPAD, HID), jnp.float32),
    )(g, s, deg, h, msg_b, lin_b, ln_g, ln_b)


def _k3_body(h_ref, w1_ref, b1_ref, w2_ref, b2_ref, w3_ref, b3_ref, o_ref):
    d = jnp.maximum(jnp.dot(h_ref[...], w1_ref[...],
                            preferred_element_type=jnp.float32)
                    + b1_ref[...], 0.0)
    d = jnp.maximum(jnp.dot(d, w2_ref[...],
                            preferred_element_type=jnp.float32)
                    + b2_ref[...], 0.0)
    o = jnp.dot(d, w3_ref[...], preferred_element_type=jnp.float32) \
        + b3_ref[...]
    o_ref[...] = jnp.tanh(o) * np.pi


def _decoder(h, w1, b1, w2, b2, w3p, b3p):
    grid = (NPAD // BM,)
    return pl.pallas_call(
        _k3_body,
        grid=grid,
        in_specs=[
            pl.BlockSpec((BM, HID), lambda r: (r, 0)),
            pl.BlockSpec((HID, HID), lambda r: (0, 0)),
            pl.BlockSpec((1, HID), lambda r: (0, 0)),
            pl.BlockSpec((HID, HID // 2), lambda r: (0, 0)),
            pl.BlockSpec((1, HID // 2), lambda r: (0, 0)),
            pl.BlockSpec((HID // 2, CW), lambda r: (0, 0)),
            pl.BlockSpec((1, CW), lambda r: (0, 0)),
        ],
        out_specs=pl.BlockSpec((BM, CW), lambda r: (r, 0)),
        out_shape=jax.ShapeDtypeStruct((NPAD, CW), jnp.float32),
    )(h, w1, b1, w2, b2, w3p, b3p)


def _sc_scatter(nblk, gflat, gidx, dst2, zeros):
    mesh = plsc.VectorSubcoreMesh(core_axis_name="c", subcore_axis_name="s")

    @functools.partial(
        pl.kernel,
        out_type=jax.ShapeDtypeStruct((NCHUNK, NPAD, CW), jnp.float32),
        mesh=mesh,
        scratch_types=[
            pltpu.VMEM((nblk, 128), jnp.int32),
            pltpu.VMEM((nblk, 128), jnp.int32),
            pltpu.VMEM((128, CW), jnp.float32),
            pltpu.VMEM_SHARED((NPAD, CW), jnp.float32),
            pltpu.SemaphoreType.DMA,
        ],
    )
    def scatter_kernel(gflat_hbm, gidx_hbm, dst_hbm, zeros_hbm, out_hbm,
                       gidx_v, dst_v, rows_v, acc_sh, sem):
        cid = lax.axis_index("c")
        tid = lax.axis_index("s")
        pltpu.sync_copy(dst_hbm.at[tid], dst_v)
        for p in range(NCHUNK // 2):
            c = cid * (NCHUNK // 2) + p
            pltpu.sync_copy(zeros_hbm.at[pl.ds(tid * RPT, RPT)],
                            acc_sh.at[pl.ds(tid * RPT, RPT)])
            pltpu.sync_copy(gidx_hbm.at[c].at[tid], gidx_v)
            plsc.subcore_barrier()

            def body(j, carry):
                pltpu.async_copy(gflat_hbm.at[gidx_v.at[j]], rows_v,
                                 sem).wait()
                pltpu.sync_copy(rows_v, acc_sh.at[dst_v.at[j]], add=True)
                return carry

            lax.fori_loop(0, nblk, body, 0)
            plsc.subcore_barrier()
            pltpu.sync_copy(acc_sh.at[pl.ds(tid * RPT, RPT)],
                            out_hbm.at[c].at[pl.ds(tid * RPT, RPT)])
            plsc.subcore_barrier()

    return scatter_kernel(gflat, gidx, dst2, zeros)


def _sc_degree(nblk, dst2, ones, zeros):
    mesh = plsc.VectorSubcoreMesh(core_axis_name="c", subcore_axis_name="s")

    @functools.partial(
        pl.kernel,
        out_type=jax.ShapeDtypeStruct((NPAD, CW), jnp.float32),
        mesh=mesh,
        scratch_types=[
            pltpu.VMEM((nblk, 128), jnp.int32),
            pltpu.VMEM((128, CW), jnp.float32),
            pltpu.VMEM_SHARED((NPAD, CW), jnp.float32),
        ],
    )
    def degree_kernel(dst_hbm, ones_hbm, zeros_hbm, out_hbm,
                      dst_v, ones_v, acc_sh):
        cid = lax.axis_index("c")
        tid = lax.axis_index("s")

        @pl.when(cid == 0)
        def _():
            pltpu.sync_copy(dst_hbm.at[tid], dst_v)
            pltpu.sync_copy(ones_hbm, ones_v)
            pltpu.sync_copy(zeros_hbm.at[pl.ds(tid * RPT, RPT)],
                            acc_sh.at[pl.ds(tid * RPT, RPT)])
            plsc.subcore_barrier()

            def body(j, carry):
                pltpu.sync_copy(ones_v, acc_sh.at[dst_v.at[j]], add=True)
                return carry

            lax.fori_loop(0, nblk, body, 0)
            plsc.subcore_barrier()
            pltpu.sync_copy(acc_sh.at[pl.ds(tid * RPT, RPT)],
                            out_hbm.at[pl.ds(tid * RPT, RPT)])

    return degree_kernel(dst2, ones, zeros)


def kernel(params, sequence, edge_index):
    p = params
    src = edge_index[0]
    dst = edge_index[1]
    e = src.shape[0]
    epad = ((e + NTILES * 128 - 1) // (NTILES * 128)) * (NTILES * 128)
    nblk = epad // (NTILES * 128)

    srcp = jnp.concatenate(
        [src, jnp.zeros((epad - e,), jnp.int32)])
    dstp = jnp.concatenate(
        [dst, jnp.full((epad - e,), N, jnp.int32)])
    grow = GROWS * srcp + 2 * HID // CW
    gidx = (grow[None, :]
            + jnp.arange(NCHUNK, dtype=jnp.int32)[:, None]).reshape(
                NCHUNK, NTILES, nblk, 128)
    dst2 = dstp.reshape(NTILES, nblk, 128)
    zeros = jnp.zeros((NPAD, CW), jnp.float32)
    ones = jnp.ones((128, CW), jnp.float32)

    seq2 = jnp.concatenate(
        [sequence, jnp.full((NPAD - N,), 31, jnp.int32)])[:, None]
    pos = jnp.concatenate(
        [p["pos_emb"], jnp.zeros((NPAD - N, EMB), jnp.float32)])
    aa32 = jnp.concatenate(
        [p["aa_emb"], jnp.zeros((12, EMB), jnp.float32)])

    deg = _sc_degree(nblk, dst2, ones, zeros)

    h = _input_proj(seq2, pos, aa32, p["in_W"][:EMB], p["in_W"][EMB:],
                    p["in_b"][None, :])

    for i, lp in enumerate(p["layers"]):
        w3 = jnp.concatenate(
            [lp["lin_W"], lp["msg_W"][:HID], lp["msg_W"][HID:]], axis=1)
        g = _fused_matmul(h, w3)
        s = _sc_scatter(nblk, g.reshape(NPAD * GROWS, CW), gidx, dst2, zeros)
        h = _combine(i > 0, g, s, deg, h, lp["msg_b"][None, :],
                     lp["lin_b"][None, :], lp["ln_g"][None, :],
                     lp["ln_b"][None, :])

    w3p = jnp.concatenate(
        [p["dec3_W"], jnp.zeros((HID // 2, CW - 2), jnp.float32)], axis=1)
    b3p = jnp.concatenate([p["dec3_b"], jnp.zeros((CW - 2,), jnp.float32)])
    out = _decoder(h, p["dec1_W"], p["dec1_b"][None, :],
                   p["dec2_W"], p["dec2_b"][None, :], w3p, b3p[None, :])
    return out[:N, :2]
